# baseline (speedup 1.0000x reference)
import functools

import jax
import jax.numpy as jnp
from jax import lax
from jax.experimental import pallas as pl
from jax.experimental.pallas import tpu as pltpu
from jax.experimental.pallas import tpu_sc as plsc

N = 100000
C = 128
L = 16
NW = 32
G = N // L
PG = -(-G // NW)
CG = 14
CH = CG * L
NCH = PG // CG


def _body(x_hbm, idx_hbm, out_hbm, x_v, out_v, idx_v, t64_v, sq_v):
    wid = lax.axis_index("s") * 2 + lax.axis_index("c")
    tstart = jnp.minimum(wid * PG, G - PG) * L

    iota16 = lax.iota(jnp.int32, L)

    pltpu.sync_copy(x_hbm.at[:, pl.ds(0, 64)], t64_v)

    def sq_body(e, carry):
        ev = jnp.full((L,), e, jnp.int32)
        for cg in range(C // L):
            col = plsc.load_gather(t64_v, [iota16 + cg * L, ev])
            sq_v[e, pl.ds(cg * L, L)] = col * col
        return carry

    lax.fori_loop(0, 64, sq_body, 0)

    def chunk_body(k, carry):
        rbase = tstart + k * CH
        pltpu.sync_copy(idx_hbm.at[pl.ds(rbase, CH)], idx_v)
        pltpu.sync_copy(x_hbm.at[:, pl.ds(rbase, CH)], x_v)

        def group_body(ig, c2):
            ev = jnp.clip(idx_v[pl.ds(ig * L, L)], 0, 63)
            rvec = ig * L + iota16

            def c_body(c, c3):
                cv = jnp.full((L,), c, jnp.int32)
                xv = x_v[c, pl.ds(ig * L, L)]
                sv = plsc.load_gather(sq_v, [ev, cv])
                plsc.store_scatter(out_v, [rvec, cv], xv * sv)
                return c3

            lax.fori_loop(0, C, c_body, 0)
            return c2

        lax.fori_loop(0, CG, group_body, 0)
        pltpu.sync_copy(out_v, out_hbm.at[pl.ds(rbase, CH), :])
        return carry

    lax.fori_loop(0, NCH, chunk_body, 0)


@jax.jit
def kernel(inp1, inp2):
    idx32 = inp2.reshape(-1).astype(jnp.int32)
    mesh = plsc.VectorSubcoreMesh(core_axis_name="c", subcore_axis_name="s")
    run = functools.partial(
        pl.kernel,
        mesh=mesh,
        compiler_params=pltpu.CompilerParams(
            use_tc_tiling_on_sc=False, needs_layout_passes=False
        ),
        out_type=jax.ShapeDtypeStruct((N, C), jnp.float32),
        scratch_types=[
            pltpu.VMEM((C, CH), jnp.float32),
            pltpu.VMEM((CH, C), jnp.float32),
            pltpu.VMEM((CH,), jnp.int32),
            pltpu.VMEM((C, 64), jnp.float32),
            pltpu.VMEM((64, C), jnp.float32),
        ],
    )(_body)
    return run(inp1, idx32)
